# baseline (speedup 1.0000x reference)
import functools

import jax
import jax.numpy as jnp
from jax import lax
from jax.experimental import pallas as pl
from jax.experimental.pallas import tpu as pltpu
from jax.experimental.pallas import tpu_sc as plsc

_SC_INFO = plsc.get_sparse_core_info()
_NC, _NS = _SC_INFO.num_cores, _SC_INFO.num_subcores
_NW = _NC * _NS
_TPW = 2048 // _NW

_B, _D, _H, _E = 2048, 768, 2048, 8
_T = 128
_NB = _B // _T

_INTERPRET = False


def _gate_body(x_ref, gw_ref, gb_ref, top1_ref, off_ref):
    logits = jnp.dot(x_ref[...], gw_ref[...], preferred_element_type=jnp.float32)
    logits = logits + gb_ref[...]
    scores = jax.nn.softmax(logits, axis=-1)
    ids = jnp.argmax(scores, axis=-1).astype(jnp.int32)
    top1_ref[...] = ids[:, None]
    kidx = lax.broadcasted_iota(jnp.int32, (_B, 16), 1)
    less = (ids[:, None] < kidx).astype(jnp.int32)
    off_ref[...] = jnp.sum(less, axis=0)[None, :]


def _gating(x, gate_w, gate_b):
    return pl.pallas_call(
        _gate_body,
        out_shape=(jax.ShapeDtypeStruct((_B, 1), jnp.int32),
                   jax.ShapeDtypeStruct((1, 16), jnp.int32)),
        interpret=_INTERPRET,
    )(x, gate_w, gate_b)


def _worker_id():
    return lax.axis_index("s") * _NC + lax.axis_index("c")


def _dispatch_body(x_hbm, dest_hbm, xs_hbm, dest_v, xr_v, sem):
    w = _worker_id()
    tok0 = w * _TPW
    pltpu.sync_copy(dest_hbm.at[pl.ds(tok0, _TPW)], dest_v)
    pltpu.sync_copy(x_hbm.at[pl.ds(tok0, _TPW)], xr_v)
    pltpu.async_copy(xr_v, xs_hbm.at[dest_v], sem).wait()


def _dispatch(x, dest):
    mesh = plsc.VectorSubcoreMesh(core_axis_name="c", subcore_axis_name="s",
                                  num_cores=_NC)
    f = functools.partial(
        pl.kernel,
        out_type=jax.ShapeDtypeStruct((_B, _D), jnp.float32),
        mesh=mesh,
        scratch_types=[
            pltpu.VMEM((_TPW,), jnp.int32),
            pltpu.VMEM((_TPW, _D), jnp.float32),
            pltpu.SemaphoreType.DMA,
        ],
    )(_dispatch_body)
    return f(x, dest)


def _combine_body(ys_hbm, dest_hbm, out_hbm, dest_v, yr_v, sem):
    w = _worker_id()
    tok0 = w * _TPW
    pltpu.sync_copy(dest_hbm.at[pl.ds(tok0, _TPW)], dest_v)
    pltpu.async_copy(ys_hbm.at[dest_v], yr_v, sem).wait()
    pltpu.sync_copy(yr_v, out_hbm.at[pl.ds(tok0, _TPW)])


def _combine(ys, dest):
    mesh = plsc.VectorSubcoreMesh(core_axis_name="c", subcore_axis_name="s",
                                  num_cores=_NC)
    f = functools.partial(
        pl.kernel,
        out_type=jax.ShapeDtypeStruct((_B, _D), jnp.float32),
        mesh=mesh,
        scratch_types=[
            pltpu.VMEM((_TPW,), jnp.int32),
            pltpu.VMEM((_TPW, _D), jnp.float32),
            pltpu.SemaphoreType.DMA,
        ],
    )(_combine_body)
    return f(ys, dest)


def _ffn_body(off_ref, x_hbm, w1_ref, b1_ref, w2_ref, b2_ref, out_hbm,
              x_scr, out_scr, sem):
    e = pl.program_id(0)
    s0 = off_ref[e]
    s1 = off_ref[e + 1]

    @pl.when(e == 0)
    def _():
        cp = pltpu.make_async_copy(x_hbm, x_scr, sem)
        cp.start()
        cp.wait()

    w1e = w1_ref[0].astype(jnp.bfloat16)
    w2e = w2_ref[0].astype(jnp.bfloat16)

    def body(b, carry):
        active = (s1 > s0) & (b * _T < s1) & ((b + 1) * _T > s0)

        @pl.when(active)
        def _():
            rows = pl.ds(b * _T, _T)
            xb = x_scr[rows, :].astype(jnp.bfloat16)
            h = jnp.dot(xb, w1e, preferred_element_type=jnp.float32)
            h = jnp.maximum(h + b1_ref[0], 0.0).astype(jnp.bfloat16)
            y = jnp.dot(h, w2e, preferred_element_type=jnp.float32) + b2_ref[0]
            ridx = lax.broadcasted_iota(jnp.int32, (_T, 1), 0) + b * _T
            mask = (ridx >= s0) & (ridx < s1)
            out_scr[rows, :] = jnp.where(mask, y, out_scr[rows, :])

        return carry

    lax.fori_loop(0, _NB, body, 0)

    @pl.when(e == _E - 1)
    def _():
        cp = pltpu.make_async_copy(out_scr, out_hbm, sem)
        cp.start()
        cp.wait()


def _ffn(offsets, x_sorted, w1, b1, w2, b2):
    grid_spec = pltpu.PrefetchScalarGridSpec(
        num_scalar_prefetch=1,
        grid=(_E,),
        in_specs=[
            pl.BlockSpec(memory_space=pl.ANY),
            pl.BlockSpec((1, _D, _H), lambda e, s: (e, 0, 0)),
            pl.BlockSpec((1, 1, _H), lambda e, s: (e, 0, 0)),
            pl.BlockSpec((1, _H, _D), lambda e, s: (e, 0, 0)),
            pl.BlockSpec((1, 1, _D), lambda e, s: (e, 0, 0)),
        ],
        out_specs=pl.BlockSpec(memory_space=pl.ANY),
        scratch_shapes=[
            pltpu.VMEM((_B, _D), jnp.float32),
            pltpu.VMEM((_B, _D), jnp.float32),
            pltpu.SemaphoreType.DMA,
        ],
    )
    return pl.pallas_call(
        _ffn_body,
        grid_spec=grid_spec,
        out_shape=jax.ShapeDtypeStruct((_B, _D), jnp.float32),
        compiler_params=pltpu.CompilerParams(
            dimension_semantics=("arbitrary",)),
        interpret=_INTERPRET,
    )(offsets, x_sorted, w1, b1, w2, b2)


def kernel(x, gate_w, gate_b, w1, b1, w2, b2):
    top1_2d, offs = _gating(x, gate_w, gate_b.reshape(1, _E))
    offs16 = offs.reshape(16)
    top1 = top1_2d.reshape(_B)
    oh = (top1[:, None] == jnp.arange(_E, dtype=jnp.int32)).astype(jnp.int32)
    ranks = jnp.cumsum(oh, axis=0)
    rank_t = jnp.take_along_axis(ranks, top1[:, None], axis=1)[:, 0]
    dest = (offs16[top1] + rank_t - 1).astype(jnp.int32)
    x_sorted = _dispatch(x, dest)
    out_sorted = _ffn(offs16, x_sorted, w1,
                      b1.reshape(_E, 1, _H), w2, b2.reshape(_E, 1, _D))
    return _combine(out_sorted, dest)


# speedup vs baseline: 1.2077x; 1.2077x over previous
import functools

import jax
import jax.numpy as jnp
from jax import lax
from jax.experimental import pallas as pl
from jax.experimental.pallas import tpu as pltpu
from jax.experimental.pallas import tpu_sc as plsc

_SC_INFO = plsc.get_sparse_core_info()
_NC, _NS = _SC_INFO.num_cores, _SC_INFO.num_subcores
_NW = _NC * _NS
_TPW = 2048 // _NW

_B, _D, _H, _E = 2048, 768, 2048, 8
_T = 128
_NB = _B // _T


def _gate_body(x_ref, gw_ref, gb_ref, top1_ref, off_ref):
    logits = jnp.dot(x_ref[...], gw_ref[...], preferred_element_type=jnp.float32)
    logits = logits + gb_ref[...]
    scores = jax.nn.softmax(logits, axis=-1)
    ids = jnp.argmax(scores, axis=-1).astype(jnp.int32)
    top1_ref[...] = ids[:, None]
    kidx = lax.broadcasted_iota(jnp.int32, (_B, 16), 1)
    less = (ids[:, None] < kidx).astype(jnp.int32)
    off_ref[...] = jnp.sum(less, axis=0)[None, :]


def _gating(x, gate_w, gate_b):
    return pl.pallas_call(
        _gate_body,
        out_shape=(jax.ShapeDtypeStruct((_B, 1), jnp.int32),
                   jax.ShapeDtypeStruct((1, 16), jnp.int32)),
    )(x, gate_w, gate_b)


def _worker_id():
    return lax.axis_index("s") * _NC + lax.axis_index("c")


def _dispatch_body(x_hbm, sidx_hbm, xs_hbm, sidx_v, xr_v, sem):
    w = _worker_id()
    tok0 = w * _TPW
    pltpu.sync_copy(sidx_hbm.at[pl.ds(tok0, _TPW)], sidx_v)
    pltpu.async_copy(x_hbm.at[sidx_v], xr_v, sem).wait()
    pltpu.sync_copy(xr_v, xs_hbm.at[pl.ds(tok0, _TPW)])


def _dispatch(x, sort_idx):
    mesh = plsc.VectorSubcoreMesh(core_axis_name="c", subcore_axis_name="s",
                                  num_cores=_NC)
    f = functools.partial(
        pl.kernel,
        out_type=jax.ShapeDtypeStruct((_B, _D), jnp.float32),
        mesh=mesh,
        scratch_types=[
            pltpu.VMEM((_TPW,), jnp.int32),
            pltpu.VMEM((_TPW, _D), jnp.float32),
            pltpu.SemaphoreType.DMA,
        ],
    )(_dispatch_body)
    return f(x, sort_idx)


def _combine_body(ys_hbm, sidx_hbm, out_hbm, sidx_v, yr_v, sem):
    w = _worker_id()
    tok0 = w * _TPW
    pltpu.sync_copy(sidx_hbm.at[pl.ds(tok0, _TPW)], sidx_v)
    pltpu.sync_copy(ys_hbm.at[pl.ds(tok0, _TPW)], yr_v)
    pltpu.async_copy(yr_v, out_hbm.at[sidx_v], sem).wait()


def _combine(ys, sort_idx):
    mesh = plsc.VectorSubcoreMesh(core_axis_name="c", subcore_axis_name="s",
                                  num_cores=_NC)
    f = functools.partial(
        pl.kernel,
        out_type=jax.ShapeDtypeStruct((_B, _D), jnp.float32),
        mesh=mesh,
        scratch_types=[
            pltpu.VMEM((_TPW,), jnp.int32),
            pltpu.VMEM((_TPW, _D), jnp.float32),
            pltpu.SemaphoreType.DMA,
        ],
    )(_combine_body)
    return f(ys, sort_idx)


def _ffn_body(off_ref, x_hbm, w1_ref, b1_ref, w2_ref, b2_ref, out_hbm,
              x_scr, out_scr, sem):
    e = pl.program_id(0)
    s0 = off_ref[e]
    s1 = off_ref[e + 1]

    @pl.when(e == 0)
    def _():
        cp = pltpu.make_async_copy(x_hbm, x_scr, sem)
        cp.start()
        cp.wait()

    w1e = w1_ref[0].astype(jnp.bfloat16)
    w2e = w2_ref[0].astype(jnp.bfloat16)

    def body(b, carry):
        active = (s1 > s0) & (b * _T < s1) & ((b + 1) * _T > s0)

        @pl.when(active)
        def _():
            rows = pl.ds(b * _T, _T)
            xb = x_scr[rows, :].astype(jnp.bfloat16)
            h = jnp.dot(xb, w1e, preferred_element_type=jnp.float32)
            h = jnp.maximum(h + b1_ref[0], 0.0).astype(jnp.bfloat16)
            y = jnp.dot(h, w2e, preferred_element_type=jnp.float32) + b2_ref[0]
            ridx = lax.broadcasted_iota(jnp.int32, (_T, 1), 0) + b * _T
            mask = (ridx >= s0) & (ridx < s1)
            out_scr[rows, :] = jnp.where(mask, y, out_scr[rows, :])

        return carry

    lax.fori_loop(0, _NB, body, 0)

    @pl.when(e == _E - 1)
    def _():
        cp = pltpu.make_async_copy(out_scr, out_hbm, sem)
        cp.start()
        cp.wait()


def _ffn(offsets, x_sorted, w1, b1, w2, b2):
    grid_spec = pltpu.PrefetchScalarGridSpec(
        num_scalar_prefetch=1,
        grid=(_E,),
        in_specs=[
            pl.BlockSpec(memory_space=pl.ANY),
            pl.BlockSpec((1, _D, _H), lambda e, s: (e, 0, 0)),
            pl.BlockSpec((1, 1, _H), lambda e, s: (e, 0, 0)),
            pl.BlockSpec((1, _H, _D), lambda e, s: (e, 0, 0)),
            pl.BlockSpec((1, 1, _D), lambda e, s: (e, 0, 0)),
        ],
        out_specs=pl.BlockSpec(memory_space=pl.ANY),
        scratch_shapes=[
            pltpu.VMEM((_B, _D), jnp.float32),
            pltpu.VMEM((_B, _D), jnp.float32),
            pltpu.SemaphoreType.DMA,
        ],
    )
    return pl.pallas_call(
        _ffn_body,
        grid_spec=grid_spec,
        out_shape=jax.ShapeDtypeStruct((_B, _D), jnp.float32),
        compiler_params=pltpu.CompilerParams(
            dimension_semantics=("arbitrary",)),
    )(offsets, x_sorted, w1, b1, w2, b2)


def kernel(x, gate_w, gate_b, w1, b1, w2, b2):
    top1_2d, offs = _gating(x, gate_w, gate_b.reshape(1, _E))
    offs16 = offs.reshape(16)
    sort_idx = jnp.argsort(top1_2d.reshape(_B)).astype(jnp.int32)
    x_sorted = _dispatch(x, sort_idx)
    out_sorted = _ffn(offs16, x_sorted, w1,
                      b1.reshape(_E, 1, _H), w2, b2.reshape(_E, 1, _D))
    return _combine(out_sorted, sort_idx)
